# baseline (speedup 1.0000x reference)
import functools

import jax
import jax.numpy as jnp
from jax import lax
from jax.experimental import pallas as pl
from jax.experimental.pallas import tpu as pltpu
from jax.experimental.pallas import tpu_sc as plsc

IMG = 384
PATCH = 16
FEAT = IMG // PATCH
NUM_PATCHES = FEAT * FEAT
CUTOFF = NUM_PATCHES // 2
NHEADS = 12
BATCH = 16
BB = 2
LANES = 16
NCH = NUM_PATCHES // LANES

_I32_MIN = -(2 ** 31)
_I32_MAXP = (1 << 31) - 1


def _sc_mask_body(att_hbm, out_hbm, rows_v, vals_v, skey_v, mask_v):
    wid = lax.axis_index("s")

    pltpu.sync_copy(att_hbm.at[wid], rows_v)

    for i in range(NCH):
        acc = rows_v[pl.ds(i * LANES, LANES)]
        for h in range(1, NHEADS):
            acc = acc + rows_v[pl.ds(h * NUM_PATCHES + i * LANES, LANES)]
        vals_v[pl.ds(i * LANES, LANES)] = acc
        bits = lax.bitcast_convert_type(acc, jnp.int32)
        skey_v[pl.ds(i * LANES, LANES)] = jnp.where(
            bits >= 0, bits, bits ^ jnp.int32(_I32_MAXP))

    def search_bit(j, tu):
        bp = lax.shift_left(jnp.full((LANES,), 1, jnp.int32),
                            jnp.int32(31) - j)
        cand_u = tu | bp
        cand_s = cand_u ^ jnp.int32(_I32_MIN)
        cnt = jnp.zeros((LANES,), jnp.int32)
        for i in range(NCH):
            sk = skey_v[pl.ds(i * LANES, LANES)]
            cnt = cnt + plsc.all_reduce_population_count(sk >= cand_s)
        return jnp.where(cnt >= CUTOFF, cand_u, tu)

    tu = lax.fori_loop(0, 32, search_bit, jnp.zeros((LANES,), jnp.int32))
    ts = tu ^ jnp.int32(_I32_MIN)
    thr_bits = jnp.where(ts >= 0, ts, ts ^ jnp.int32(_I32_MAXP))
    thr = lax.bitcast_convert_type(thr_bits, jnp.float32)

    for i in range(NCH):
        v = vals_v[pl.ds(i * LANES, LANES)]
        mask_v[pl.ds(i * LANES, LANES)] = jnp.where(
            v > thr, jnp.float32(1), jnp.float32(0))

    pltpu.sync_copy(mask_v, out_hbm.at[wid])


_sc_mask = functools.partial(
    pl.kernel,
    out_type=jax.ShapeDtypeStruct((BATCH, NUM_PATCHES), jnp.float32),
    mesh=plsc.VectorSubcoreMesh(core_axis_name="c", subcore_axis_name="s",
                                num_cores=1),
    compiler_params=pltpu.CompilerParams(needs_layout_passes=False),
    scratch_types=[
        pltpu.VMEM((NHEADS * NUM_PATCHES,), jnp.float32),
        pltpu.VMEM((NUM_PATCHES,), jnp.float32),
        pltpu.VMEM((NUM_PATCHES,), jnp.int32),
        pltpu.VMEM((NUM_PATCHES,), jnp.float32),
    ],
)(_sc_mask_body)


def _upsample_multiply(m, img):
    p_cols = jnp.where(
        jax.lax.broadcasted_iota(jnp.int32, (FEAT, IMG), 1) // PATCH
        == jax.lax.broadcasted_iota(jnp.int32, (FEAT, IMG), 0),
        1.0, 0.0).astype(jnp.float32)
    p_rows = jnp.where(
        jax.lax.broadcasted_iota(jnp.int32, (IMG, FEAT), 0) // PATCH
        == jax.lax.broadcasted_iota(jnp.int32, (IMG, FEAT), 1),
        1.0, 0.0).astype(jnp.float32)
    mp = jax.lax.dot_general(m, p_cols, (((1,), (0,)), ((), ())),
                             preferred_element_type=jnp.float32)
    m_full = jax.lax.dot_general(p_rows, mp, (((1,), (0,)), ((), ())),
                                 preferred_element_type=jnp.float32)
    return img * m_full[None, :, :]


def _tc_body_a(att_ref, img_ref, out_ref, mask_ref):
    b = pl.program_id(0)

    @pl.when(b == 0)
    def _prologue():
        a = jnp.sum(att_ref[...], axis=1)
        bits = jax.lax.bitcast_convert_type(a, jnp.int32)
        skey = jnp.where(bits >= 0, bits, bits ^ jnp.int32(_I32_MAXP))
        tu = jnp.zeros((a.shape[0], 1, 1), jnp.int32)
        for bit in range(31, -1, -1):
            bp = jnp.int32(_I32_MIN) if bit == 31 else jnp.int32(1 << bit)
            cand_u = tu | bp
            cand_s = cand_u ^ jnp.int32(_I32_MIN)
            cnt = jnp.sum((skey >= cand_s).astype(jnp.int32),
                          axis=(1, 2), keepdims=True)
            tu = jnp.where(cnt >= CUTOFF, cand_u, tu)
        ts = tu ^ jnp.int32(_I32_MIN)
        thr_bits = jnp.where(ts >= 0, ts, ts ^ jnp.int32(_I32_MAXP))
        thr = jax.lax.bitcast_convert_type(thr_bits, jnp.float32)
        mask_ref[...] = jnp.where(a > thr, 1.0, 0.0).astype(jnp.float32)

    for j in range(BB):
        out_ref[j] = _upsample_multiply(mask_ref[BB * b + j], img_ref[j])


def _tc_body_b(buf_ref, mask_ref, img_ref, out_ref):
    del buf_ref
    b = pl.program_id(0)
    for j in range(BB):
        out_ref[j] = _upsample_multiply(mask_ref[TSPLIT + BB * b + j],
                                        img_ref[j])


TSPLIT = 8


def kernel(x, input_images):
    B, NH = x.shape[0], x.shape[1]
    att_rows = x[:, :, 0, 1:]
    att4 = att_rows.reshape(B, NH, FEAT, FEAT)
    att_flat = att_rows.reshape(B, NH * NUM_PATCHES)
    mask24 = _sc_mask(att_flat).reshape(B, FEAT, FEAT)
    out_a = pl.pallas_call(
        _tc_body_a,
        grid=(TSPLIT // BB,),
        in_specs=[
            pl.BlockSpec((B, NH, FEAT, FEAT), lambda b: (0, 0, 0, 0)),
            pl.BlockSpec((BB, 3, IMG, IMG), lambda b: (b, 0, 0, 0)),
        ],
        out_specs=pl.BlockSpec((BB, 3, IMG, IMG), lambda b: (b, 0, 0, 0)),
        out_shape=jax.ShapeDtypeStruct(input_images.shape, input_images.dtype),
        scratch_shapes=[pltpu.VMEM((B, FEAT, FEAT), jnp.float32)],
    )(att4, input_images)
    nb = (B - TSPLIT) // BB
    return pl.pallas_call(
        _tc_body_b,
        grid=(nb,),
        in_specs=[
            pl.BlockSpec(memory_space=pl.ANY),
            pl.BlockSpec((B, FEAT, FEAT), lambda b: (0, 0, 0)),
            pl.BlockSpec((BB, 3, IMG, IMG),
                         lambda b: (b + TSPLIT // BB, 0, 0, 0)),
        ],
        out_specs=pl.BlockSpec((BB, 3, IMG, IMG),
                               lambda b: (b + TSPLIT // BB, 0, 0, 0)),
        out_shape=jax.ShapeDtypeStruct(input_images.shape, input_images.dtype),
        input_output_aliases={0: 0},
    )(out_a, mask24, input_images)


# speedup vs baseline: 1.2177x; 1.2177x over previous
import functools

import jax
import jax.numpy as jnp
from jax import lax
from jax.experimental import pallas as pl
from jax.experimental.pallas import tpu as pltpu
from jax.experimental.pallas import tpu_sc as plsc

IMG = 384
PATCH = 16
FEAT = IMG // PATCH
NUM_PATCHES = FEAT * FEAT
CUTOFF = NUM_PATCHES // 2
NHEADS = 12
BATCH = 16
TSPLIT = 12
BBA = 4
BBB = 2
LANES = 16
NCH = NUM_PATCHES // LANES

_I32_MIN = -(2 ** 31)
_I32_MAXP = (1 << 31) - 1


def _sc_mask_body(att_hbm, out_hbm, rows_v, vals_v, skey_v, mask_v):
    wid = lax.axis_index("s")

    pltpu.sync_copy(att_hbm.at[wid], rows_v)

    for i in range(NCH):
        acc = rows_v[0, pl.ds(i * LANES, LANES)]
        for h in range(1, NHEADS):
            acc = acc + rows_v[h, pl.ds(i * LANES, LANES)]
        vals_v[pl.ds(i * LANES, LANES)] = acc
        bits = lax.bitcast_convert_type(acc, jnp.int32)
        skey_v[pl.ds(i * LANES, LANES)] = jnp.where(
            bits >= 0, bits, bits ^ jnp.int32(_I32_MAXP))

    def search_bit(j, tu):
        bp = lax.shift_left(jnp.full((LANES,), 1, jnp.int32),
                            jnp.int32(31) - j)
        cand_u = tu | bp
        cand_s = cand_u ^ jnp.int32(_I32_MIN)
        cnt = jnp.zeros((LANES,), jnp.int32)
        for i in range(NCH):
            sk = skey_v[pl.ds(i * LANES, LANES)]
            cnt = cnt + plsc.all_reduce_population_count(sk >= cand_s)
        return jnp.where(cnt >= CUTOFF, cand_u, tu)

    tu = lax.fori_loop(0, 32, search_bit, jnp.zeros((LANES,), jnp.int32))
    ts = tu ^ jnp.int32(_I32_MIN)
    thr_bits = jnp.where(ts >= 0, ts, ts ^ jnp.int32(_I32_MAXP))
    thr = lax.bitcast_convert_type(thr_bits, jnp.float32)

    for i in range(NCH):
        v = vals_v[pl.ds(i * LANES, LANES)]
        mask_v[pl.ds(i * LANES, LANES)] = jnp.where(
            v > thr, jnp.float32(1), jnp.float32(0))

    pltpu.sync_copy(mask_v, out_hbm.at[wid])


_sc_mask = functools.partial(
    pl.kernel,
    out_type=jax.ShapeDtypeStruct((BATCH, NUM_PATCHES), jnp.float32),
    mesh=plsc.VectorSubcoreMesh(core_axis_name="c", subcore_axis_name="s",
                                num_cores=1),
    compiler_params=pltpu.CompilerParams(needs_layout_passes=False),
    scratch_types=[
        pltpu.VMEM((NHEADS, NUM_PATCHES), jnp.float32),
        pltpu.VMEM((NUM_PATCHES,), jnp.float32),
        pltpu.VMEM((NUM_PATCHES,), jnp.int32),
        pltpu.VMEM((NUM_PATCHES,), jnp.float32),
    ],
)(_sc_mask_body)


def _upsample_multiply(m, img):
    p_cols = jnp.where(
        jax.lax.broadcasted_iota(jnp.int32, (FEAT, IMG), 1) // PATCH
        == jax.lax.broadcasted_iota(jnp.int32, (FEAT, IMG), 0),
        1.0, 0.0).astype(jnp.float32)
    p_rows = jnp.where(
        jax.lax.broadcasted_iota(jnp.int32, (IMG, FEAT), 0) // PATCH
        == jax.lax.broadcasted_iota(jnp.int32, (IMG, FEAT), 1),
        1.0, 0.0).astype(jnp.float32)
    mp = jax.lax.dot_general(m, p_cols, (((1,), (0,)), ((), ())),
                             preferred_element_type=jnp.float32)
    m_full = jax.lax.dot_general(p_rows, mp, (((1,), (0,)), ((), ())),
                                 preferred_element_type=jnp.float32)
    return img * m_full[None, :, :]


def _tc_body_a(att_ref, img_ref, out_ref, mask_ref):
    b = pl.program_id(0)

    @pl.when(b == 0)
    def _prologue():
        a = jnp.sum(att_ref[...], axis=1)
        bits = jax.lax.bitcast_convert_type(a, jnp.int32)
        skey = jnp.where(bits >= 0, bits, bits ^ jnp.int32(_I32_MAXP))
        tu = jnp.zeros((a.shape[0], 1), jnp.int32)
        for bit in range(31, -1, -1):
            bp = jnp.int32(_I32_MIN) if bit == 31 else jnp.int32(1 << bit)
            cand_u = tu | bp
            cand_s = cand_u ^ jnp.int32(_I32_MIN)
            cnt = jnp.sum((skey >= cand_s).astype(jnp.int32),
                          axis=1, keepdims=True)
            tu = jnp.where(cnt >= CUTOFF, cand_u, tu)
        ts = tu ^ jnp.int32(_I32_MIN)
        thr_bits = jnp.where(ts >= 0, ts, ts ^ jnp.int32(_I32_MAXP))
        thr = jax.lax.bitcast_convert_type(thr_bits, jnp.float32)
        mask_ref[...] = jnp.where(a > thr, 1.0, 0.0).reshape(
            a.shape[0], FEAT, FEAT)

    for j in range(BBA):
        out_ref[j] = _upsample_multiply(mask_ref[BBA * b + j], img_ref[j])


def _tc_body_b(buf_ref, mask_ref, img_ref, out_ref, m24_ref):
    del buf_ref
    b = pl.program_id(0)

    @pl.when(b == 0)
    def _():
        m24_ref[...] = mask_ref[...].reshape(BATCH, FEAT, FEAT)

    for j in range(BBB):
        out_ref[j] = _upsample_multiply(m24_ref[TSPLIT + BBB * b + j],
                                        img_ref[j])


def kernel(x, input_images):
    B, NH = x.shape[0], x.shape[1]
    att = x[:, :, 0, 1:]
    mask_sc = _sc_mask(att)
    out_a = pl.pallas_call(
        _tc_body_a,
        grid=(TSPLIT // BBA,),
        in_specs=[
            pl.BlockSpec((B, NH, NUM_PATCHES), lambda b: (0, 0, 0)),
            pl.BlockSpec((BBA, 3, IMG, IMG), lambda b: (b, 0, 0, 0)),
        ],
        out_specs=pl.BlockSpec((BBA, 3, IMG, IMG), lambda b: (b, 0, 0, 0)),
        out_shape=jax.ShapeDtypeStruct(input_images.shape, input_images.dtype),
        scratch_shapes=[pltpu.VMEM((B, FEAT, FEAT), jnp.float32)],
    )(att, input_images)
    return pl.pallas_call(
        _tc_body_b,
        grid=((B - TSPLIT) // BBB,),
        in_specs=[
            pl.BlockSpec(memory_space=pl.ANY),
            pl.BlockSpec((B, NUM_PATCHES), lambda b: (0, 0)),
            pl.BlockSpec((BBB, 3, IMG, IMG),
                         lambda b: (b + TSPLIT // BBB, 0, 0, 0)),
        ],
        out_specs=pl.BlockSpec((BBB, 3, IMG, IMG),
                               lambda b: (b + TSPLIT // BBB, 0, 0, 0)),
        out_shape=jax.ShapeDtypeStruct(input_images.shape, input_images.dtype),
        input_output_aliases={0: 0},
        scratch_shapes=[pltpu.VMEM((BATCH, FEAT, FEAT), jnp.float32)],
    )(out_a, mask_sc, input_images)
